# baseline (speedup 1.0000x reference)
import functools

import jax
import jax.numpy as jnp
from jax import lax
from jax.experimental import pallas as pl
from jax.experimental.pallas import tpu as pltpu
from jax.experimental.pallas import tpu_sc as plsc

CTX = 200
EMBED = 64
VOCAB = 1000000

_RING = 4
_NW = 32
_SLOTS = (CTX + _NW - 1) // _NW


def _sc_gather_sum_body(idx_hbm, tabt_hbm, out_hbm, idx_vm, blks, acc_v, *sems):
    c = lax.axis_index("c")
    t = lax.axis_index("s")
    wid = c * 16 + t

    pltpu.sync_copy(idx_hbm, idx_vm.at[pl.ds(0, CTX)])

    def get_v(j):
        chunk = idx_vm[pl.ds(32 * j + 16 * c, 16)]
        sel = lax.iota(jnp.int32, 16) == t
        return jnp.sum(jnp.where(sel, chunk, 0))

    def col_base(v):
        return pl.multiple_of(v - (v & 127), 128)

    def issue(j, b):
        pltpu.make_async_copy(
            tabt_hbm.at[:, pl.ds(col_base(get_v(j)), 128)], blks.at[b], sems[b]
        ).start()

    def active(j):
        return (wid + 32 * j < CTX) if 32 * j + 31 >= CTX else None

    def when_active(j, fn):
        a = active(j)
        if a is None:
            fn()
        else:
            pl.when(a)(fn)

    for b in range(min(_RING, _SLOTS)):
        when_active(b, lambda b=b: issue(b, b))

    acc = [jnp.zeros((16,), jnp.float32) for _ in range(4)]
    for j in range(_SLOTS):
        b = j % _RING
        when_active(j, lambda b=b: pltpu.make_async_copy(
            tabt_hbm.at[:, pl.ds(0, 128)], blks.at[b], sems[b]).wait())
        v = get_v(j)
        o = v - col_base(v)
        cols = jnp.full((16,), o, jnp.int32)
        a = active(j)
        for k in range(4):
            rows = lax.iota(jnp.int32, 16) + 16 * k
            g = plsc.load_gather(blks.at[b], [rows, cols])
            acc[k] = acc[k] + g if a is None else acc[k] + jnp.where(a, g, 0.0)
        if j + _RING < _SLOTS:
            when_active(j + _RING, lambda j=j, b=b: issue(j + _RING, b))

    for k in range(4):
        acc_v[k, :] = acc[k]
    pltpu.sync_copy(acc_v, out_hbm.at[wid])


@jax.jit
def _sc_gather_sum(inputs, tab_t):
    mesh = plsc.VectorSubcoreMesh(core_axis_name="c", subcore_axis_name="s")
    return pl.kernel(
        _sc_gather_sum_body,
        out_type=jax.ShapeDtypeStruct((_NW, 4, 16), jnp.float32),
        mesh=mesh,
        scratch_types=[
            pltpu.VMEM((32 * _SLOTS, ), jnp.int32),
            pltpu.VMEM((_RING, EMBED, 128), jnp.float32),
            pltpu.VMEM((4, 16), jnp.float32),
        ] + [pltpu.SemaphoreType.DMA] * _RING,
        compiler_params=pltpu.CompilerParams(needs_layout_passes=False),
    )(inputs, tab_t)


_SC_COLS = 96
_SC_VPT = _SC_COLS * 128
_SC_V = _SC_VPT * 32
_SC_VSTART = (7812 - 32 * _SC_COLS) * 128
_CHUNK = 512
_NCHUNK = _SC_VPT // _CHUNK


def _sc_matvec_body(p_hbm, wt_hbm, b_hbm, out_hbm, pv, ebv, bv, ov, blks,
                    sem0, sem1):
    c = lax.axis_index("c")
    t = lax.axis_index("s")
    wid = c * 16 + t
    vbase = _SC_VSTART + wid * _SC_VPT
    sems = (sem0, sem1)

    pltpu.sync_copy(p_hbm, pv)
    bcp = pltpu.make_async_copy(
        b_hbm.at[pl.ds(pl.multiple_of(vbase, 128), _SC_VPT)], bv, sem0)
    bcp.start()
    for k in range(4):
        e_k = jnp.zeros((16,), jnp.float32)
        for r in range(32):
            e_k = e_k + pv[r, k, :]
        for l in range(16):
            d = 16 * k + l
            s = jnp.sum(jnp.where(lax.iota(jnp.int32, 16) == l, e_k, 0.0))
            ebv[d, :] = jnp.full((16,), s, jnp.float32)
    bcp.wait()

    def issue(chunk, b):
        off = pl.multiple_of(vbase + chunk * _CHUNK, 128)
        pltpu.make_async_copy(
            wt_hbm.at[:, pl.ds(off, _CHUNK)], blks.at[b], sems[b]).start()

    issue(0, 0)
    issue(1, 1)

    def pair(p, carry):
        for sub in range(2):
            chunk = 2 * p + sub
            pltpu.make_async_copy(
                wt_hbm.at[:, pl.ds(0, _CHUNK)], blks.at[sub], sems[sub]).wait()

            def dbody(d, accs):
                eb = ebv[d, :]
                return tuple(
                    accs[h] + blks[sub, d, pl.ds(16 * h, 16)] * eb
                    for h in range(32)
                )

            zero = jnp.zeros((16,), jnp.float32)
            accs = lax.fori_loop(0, EMBED, dbody, (zero,) * 32)
            for h in range(32):
                off = chunk * _CHUNK + 16 * h
                ov[pl.ds(off, 16)] = accs[h] + bv[pl.ds(off, 16)]

            @pl.when(p < _NCHUNK // 2 - 1)
            def _():
                issue(chunk + 2, sub)
        return carry

    lax.fori_loop(0, _NCHUNK // 2, pair, 0)
    pltpu.sync_copy(ov, out_hbm.at[pl.ds(wid * _SC_VPT, _SC_VPT)])


@jax.jit
def _sc_matvec(partials, W_t, b):
    mesh = plsc.VectorSubcoreMesh(core_axis_name="c", subcore_axis_name="s")
    return pl.kernel(
        _sc_matvec_body,
        out_type=jax.ShapeDtypeStruct((_SC_V,), jnp.float32),
        mesh=mesh,
        scratch_types=[
            pltpu.VMEM((32, 4, 16), jnp.float32),
            pltpu.VMEM((EMBED, 16), jnp.float32),
            pltpu.VMEM((_SC_VPT,), jnp.float32),
            pltpu.VMEM((_SC_VPT,), jnp.float32),
            pltpu.VMEM((2, EMBED, _CHUNK), jnp.float32),
            pltpu.SemaphoreType.DMA,
            pltpu.SemaphoreType.DMA,
        ],
        compiler_params=pltpu.CompilerParams(needs_layout_passes=False),
    )(partials, W_t, b)


_BLKV = 32768


def _tc_matvec_body(p_ref, wt_ref, b_ref, o_ref):
    e = jnp.sum(p_ref[:], axis=0, keepdims=True)
    e8 = jnp.broadcast_to(e, (8, EMBED))
    acc = lax.dot_general(
        e8, wt_ref[:], (((1,), (0,)), ((), ())),
        preferred_element_type=jnp.float32,
    )
    o_ref[:] = acc[0] + b_ref[:]


@jax.jit
def _tc_matvec(partials, W_t, b):
    nhead = _SC_VSTART // _BLKV + 1
    nlast = pl.cdiv(VOCAB, _BLKV) - 1
    vmap = lambda i: jnp.where(i == nhead, nlast, i)
    return pl.pallas_call(
        _tc_matvec_body,
        grid=(nhead + 1,),
        in_specs=[
            pl.BlockSpec((32, EMBED), lambda i: (0, 0)),
            pl.BlockSpec((EMBED, _BLKV), lambda i: (0, vmap(i))),
            pl.BlockSpec((_BLKV,), lambda i: (vmap(i),)),
        ],
        out_specs=pl.BlockSpec((_BLKV,), lambda i: (vmap(i),)),
        out_shape=jax.ShapeDtypeStruct((VOCAB,), jnp.float32),
    )(partials, W_t, b)


def kernel(inputs, emb_table, W, b):
    partials = _sc_gather_sum(inputs, emb_table.T)
    p = partials.reshape(_NW, EMBED)
    W_t = W.T
    sc_out = _sc_matvec(partials, W_t, b)
    tc_out = _tc_matvec(p, W_t, b)
    return lax.dynamic_update_slice(tc_out, sc_out, (_SC_VSTART,))


# speedup vs baseline: 1.2113x; 1.2113x over previous
import functools

import jax
import jax.numpy as jnp
from jax import lax
from jax.experimental import pallas as pl
from jax.experimental.pallas import tpu as pltpu
from jax.experimental.pallas import tpu_sc as plsc

CTX = 200
EMBED = 64
VOCAB = 1000000

_RING = 7
_NW = 32
_SLOTS = (CTX + _NW - 1) // _NW


def _sc_gather_sum_body(idx_hbm, tabt_hbm, out_hbm, idx_vm, blks, acc_v, *sems):
    c = lax.axis_index("c")
    t = lax.axis_index("s")
    wid = c * 16 + t

    pltpu.sync_copy(idx_hbm, idx_vm.at[pl.ds(0, CTX)])

    def get_v(j):
        chunk = idx_vm[pl.ds(32 * j + 16 * c, 16)]
        sel = lax.iota(jnp.int32, 16) == t
        return jnp.sum(jnp.where(sel, chunk, 0))

    def col_base(v):
        return pl.multiple_of(v - (v & 127), 128)

    def issue(j, b):
        pltpu.make_async_copy(
            tabt_hbm.at[:, pl.ds(col_base(get_v(j)), 128)], blks.at[b], sems[b]
        ).start()

    def active(j):
        return (wid + 32 * j < CTX) if 32 * j + 31 >= CTX else None

    def when_active(j, fn):
        a = active(j)
        if a is None:
            fn()
        else:
            pl.when(a)(fn)

    for b in range(min(_RING, _SLOTS)):
        when_active(b, lambda b=b: issue(b, b))

    acc = [jnp.zeros((16,), jnp.float32) for _ in range(4)]
    for j in range(_SLOTS):
        b = j % _RING
        when_active(j, lambda b=b: pltpu.make_async_copy(
            tabt_hbm.at[:, pl.ds(0, 128)], blks.at[b], sems[b]).wait())
        v = get_v(j)
        o = v - col_base(v)
        cols = jnp.full((16,), o, jnp.int32)
        a = active(j)
        for k in range(4):
            rows = lax.iota(jnp.int32, 16) + 16 * k
            g = plsc.load_gather(blks.at[b], [rows, cols])
            acc[k] = acc[k] + g if a is None else acc[k] + jnp.where(a, g, 0.0)
        if j + _RING < _SLOTS:
            when_active(j + _RING, lambda j=j, b=b: issue(j + _RING, b))

    for k in range(4):
        acc_v[k, :] = acc[k]
    pltpu.sync_copy(acc_v, out_hbm.at[wid])


@jax.jit
def _sc_gather_sum(inputs, tab_t):
    mesh = plsc.VectorSubcoreMesh(core_axis_name="c", subcore_axis_name="s")
    return pl.kernel(
        _sc_gather_sum_body,
        out_type=jax.ShapeDtypeStruct((_NW, 4, 16), jnp.float32),
        mesh=mesh,
        scratch_types=[
            pltpu.VMEM((32 * _SLOTS, ), jnp.int32),
            pltpu.VMEM((_RING, EMBED, 128), jnp.float32),
            pltpu.VMEM((4, 16), jnp.float32),
        ] + [pltpu.SemaphoreType.DMA] * _RING,
        compiler_params=pltpu.CompilerParams(needs_layout_passes=False),
    )(inputs, tab_t)


_BLKV = 32768


def _tc_matvec_body(p_ref, wt_ref, b_ref, o_ref):
    e = jnp.sum(p_ref[:], axis=0, keepdims=True)
    e8 = jnp.broadcast_to(e, (8, EMBED))
    acc = lax.dot_general(
        e8, wt_ref[:], (((1,), (0,)), ((), ())),
        preferred_element_type=jnp.float32,
    )
    o_ref[:] = acc[0] + b_ref[:]


@jax.jit
def _tc_matvec(partials, W_t, b):
    nblk = pl.cdiv(VOCAB, _BLKV)
    return pl.pallas_call(
        _tc_matvec_body,
        grid=(nblk,),
        in_specs=[
            pl.BlockSpec((32, EMBED), lambda i: (0, 0)),
            pl.BlockSpec((EMBED, _BLKV), lambda i: (0, i)),
            pl.BlockSpec((_BLKV,), lambda i: (i,)),
        ],
        out_specs=pl.BlockSpec((_BLKV,), lambda i: (i,)),
        out_shape=jax.ShapeDtypeStruct((VOCAB,), jnp.float32),
    )(partials, W_t, b)


def kernel(inputs, emb_table, W, b):
    partials = _sc_gather_sum(inputs, emb_table.T)
    return _tc_matvec(partials.reshape(_NW, EMBED), W.T, b)


# speedup vs baseline: 1.2151x; 1.0032x over previous
import functools

import jax
import jax.numpy as jnp
from jax import lax
from jax.experimental import pallas as pl
from jax.experimental.pallas import tpu as pltpu
from jax.experimental.pallas import tpu_sc as plsc

CTX = 200
EMBED = 64
VOCAB = 1000000

_RING = 7
_NW = 32
_SLOTS = (CTX + _NW - 1) // _NW


def _sc_gather_sum_body(idx_hbm, tabt_hbm, out_hbm, idx_vm, blks, acc_v, *sems):
    c = lax.axis_index("c")
    t = lax.axis_index("s")
    wid = c * 16 + t

    pltpu.sync_copy(idx_hbm, idx_vm.at[pl.ds(0, CTX)])

    def get_v(j):
        chunk = idx_vm[pl.ds(32 * j + 16 * c, 16)]
        sel = lax.iota(jnp.int32, 16) == t
        return jnp.sum(jnp.where(sel, chunk, 0))

    def col_base(v):
        return pl.multiple_of(v - (v & 127), 128)

    def issue(j, b):
        pltpu.make_async_copy(
            tabt_hbm.at[:, pl.ds(col_base(get_v(j)), 128)], blks.at[b], sems[b]
        ).start()

    def active(j):
        return (wid + 32 * j < CTX) if 32 * j + 31 >= CTX else None

    def when_active(j, fn):
        a = active(j)
        if a is None:
            fn()
        else:
            pl.when(a)(fn)

    for b in range(min(_RING, _SLOTS)):
        when_active(b, lambda b=b: issue(b, b))

    acc = [jnp.zeros((16,), jnp.float32) for _ in range(4)]
    for j in range(_SLOTS):
        b = j % _RING
        when_active(j, lambda b=b: pltpu.make_async_copy(
            tabt_hbm.at[:, pl.ds(0, 128)], blks.at[b], sems[b]).wait())
        v = get_v(j)
        o = v - col_base(v)
        cols = jnp.full((16,), o, jnp.int32)
        a = active(j)
        for k in range(4):
            rows = lax.iota(jnp.int32, 16) + 16 * k
            g = plsc.load_gather(blks.at[b], [rows, cols])
            acc[k] = acc[k] + g if a is None else acc[k] + jnp.where(a, g, 0.0)
        if j + _RING < _SLOTS:
            when_active(j + _RING, lambda j=j, b=b: issue(j + _RING, b))

    for k in range(4):
        acc_v[k, :] = acc[k]
    pltpu.sync_copy(acc_v, out_hbm.at[wid])


@jax.jit
def _sc_gather_sum(inputs, tab_t):
    mesh = plsc.VectorSubcoreMesh(core_axis_name="c", subcore_axis_name="s")
    return pl.kernel(
        _sc_gather_sum_body,
        out_type=jax.ShapeDtypeStruct((_NW, 4, 16), jnp.float32),
        mesh=mesh,
        scratch_types=[
            pltpu.VMEM((32 * _SLOTS, ), jnp.int32),
            pltpu.VMEM((_RING, EMBED, 128), jnp.float32),
            pltpu.VMEM((4, 16), jnp.float32),
        ] + [pltpu.SemaphoreType.DMA] * _RING,
        compiler_params=pltpu.CompilerParams(needs_layout_passes=False),
    )(inputs, tab_t)


_BLKV = 35840


def _tc_matvec_body(p_ref, wt_ref, b_ref, o_ref):
    e = jnp.sum(p_ref[:], axis=0, keepdims=True)
    e8 = jnp.broadcast_to(e, (8, EMBED))
    acc = lax.dot_general(
        e8, wt_ref[:], (((1,), (0,)), ((), ())),
        preferred_element_type=jnp.float32,
    )
    o_ref[:] = acc[0] + b_ref[:]


@jax.jit
def _tc_matvec(partials, W_t, b):
    nblk = pl.cdiv(VOCAB, _BLKV)
    return pl.pallas_call(
        _tc_matvec_body,
        grid=(nblk,),
        in_specs=[
            pl.BlockSpec((32, EMBED), lambda i: (0, 0)),
            pl.BlockSpec((EMBED, _BLKV), lambda i: (0, i)),
            pl.BlockSpec((_BLKV,), lambda i: (i,)),
        ],
        out_specs=pl.BlockSpec((_BLKV,), lambda i: (i,)),
        out_shape=jax.ShapeDtypeStruct((VOCAB,), jnp.float32),
    )(partials, W_t, b)


def kernel(inputs, emb_table, W, b):
    partials = _sc_gather_sum(inputs, emb_table.T)
    return _tc_matvec(partials.reshape(_NW, EMBED), W.T, b)
